# Initial kernel scaffold; baseline (speedup 1.0000x reference)
import jax
import jax.numpy as jnp
from jax.experimental import pallas as pl


def kernel(x, edge_index, edge_weight, attention, W_conv_z, b_conv_z, W_lin_z, b_lin_z, W_conv_r, b_conv_r, W_lin_r, b_lin_r, W_conv_h, b_conv_h, W_lin_h, b_lin_h, W_out, b_out):
    raise NotImplementedError("write your pallas kernel here")


# speedup vs baseline: 170.9950x; 170.9950x over previous
import functools

import jax
import jax.numpy as jnp
from jax import lax
from jax.experimental import pallas as pl
from jax.experimental.pallas import tpu as pltpu
from jax.experimental.pallas import tpu_sc as plsc

N = 50000
P = 12
HID = 32
OUT = 12
E = 800000

NC = 2
NS = 16
NW = NC * NS

K = 128
CHUNKS = -(-E // (NW * K))
EPW = CHUNKS * K
EP = NW * EPW

RPT = (-(-N // NS) + 15) // 16 * 16
N3 = NS * RPT

_mesh = plsc.VectorSubcoreMesh(core_axis_name="c", subcore_axis_name="s")


@functools.partial(
    pl.kernel,
    out_type=[jax.ShapeDtypeStruct((N3,), jnp.float32),
              jax.ShapeDtypeStruct((N3,), jnp.float32)],
    mesh=_mesh,
    scratch_types=[
        pltpu.VMEM((K,), jnp.int32),
        pltpu.VMEM((K,), jnp.float32),
        pltpu.VMEM((RPT,), jnp.float32),
        pltpu.VMEM_SHARED((N3,), jnp.float32),
    ],
)
def _deg_kernel(col_hbm, ew_hbm, out0_hbm, out1_hbm, col_v, ew_v, zb, deg_sp):
    c = lax.axis_index("c")
    sid = lax.axis_index("s")
    wid = sid * NC + c

    def zero_body(i, _):
        zb[pl.ds(i * 16, 16)] = jnp.zeros((16,), jnp.float32)
        return 0
    lax.fori_loop(0, RPT // 16, zero_body, 0)
    pltpu.sync_copy(zb, deg_sp.at[pl.ds(sid * RPT, RPT)])
    plsc.subcore_barrier()

    base = wid * EPW

    def chunk_body(j, _):
        off = base + j * K
        pltpu.sync_copy(col_hbm.at[pl.ds(off, K)], col_v)
        pltpu.sync_copy(ew_hbm.at[pl.ds(off, K)], ew_v)
        pltpu.sync_copy(ew_v, deg_sp.at[col_v], add=True)
        return 0
    lax.fori_loop(0, CHUNKS, chunk_body, 0)

    plsc.subcore_barrier()
    pltpu.sync_copy(deg_sp.at[pl.ds(sid * RPT, RPT)], zb)

    @pl.when(c == 0)
    def _():
        pltpu.sync_copy(zb, out0_hbm.at[pl.ds(sid * RPT, RPT)])

    @pl.when(c == 1)
    def _():
        pltpu.sync_copy(zb, out1_hbm.at[pl.ds(sid * RPT, RPT)])


B2 = 2048


def _prescale_body(deg0_ref, deg1_ref, x_ref, dinv_ref, y_ref):
    deg = deg0_ref[...] + deg1_ref[...] + 1.0
    dinv = jnp.reshape(lax.rsqrt(deg), (B2, 1))
    dinv_ref[...] = dinv
    y_ref[...] = jnp.concatenate(
        [x_ref[...] * dinv, jnp.zeros((B2, 4), jnp.float32)], axis=1)


_prescale = pl.pallas_call(
    _prescale_body,
    grid=(-(-N // B2),),
    in_specs=[
        pl.BlockSpec((B2,), lambda i: (i,)),
        pl.BlockSpec((B2,), lambda i: (i,)),
        pl.BlockSpec((B2, P), lambda i: (i, 0)),
    ],
    out_specs=[
        pl.BlockSpec((B2, 1), lambda i: (i, 0)),
        pl.BlockSpec((B2, 16), lambda i: (i, 0)),
    ],
    out_shape=[
        jax.ShapeDtypeStruct((N, 1), jnp.float32),
        jax.ShapeDtypeStruct((N3, 16), jnp.float32),
    ],
)


@functools.partial(
    pl.kernel,
    out_type=[jax.ShapeDtypeStruct((N3, 16), jnp.float32),
              jax.ShapeDtypeStruct((N3, 16), jnp.float32)],
    mesh=_mesh,
    scratch_types=[
        pltpu.VMEM((K,), jnp.int32),
        pltpu.VMEM((K,), jnp.int32),
        pltpu.VMEM((K,), jnp.float32),
        pltpu.VMEM((K, 16), jnp.float32),
        pltpu.VMEM((RPT, 16), jnp.float32),
        pltpu.VMEM_SHARED((N3, 16), jnp.float32),
        pltpu.SemaphoreType.DMA,
    ],
    compiler_params=pltpu.CompilerParams(use_tc_tiling_on_sc=False),
)
def _spmm_kernel(row_hbm, col_hbm, ew_hbm, y_hbm, out0_hbm, out1_hbm,
                 row_v, col_v, ew_v, xr_v, zb, s_sp, sem):
    c = lax.axis_index("c")
    sid = lax.axis_index("s")
    wid = sid * NC + c

    def zero_body(i, _):
        zb[i] = jnp.zeros((16,), jnp.float32)
        return 0
    lax.fori_loop(0, RPT, zero_body, 0)
    pltpu.sync_copy(zb, s_sp.at[pl.ds(sid * RPT, RPT)])
    plsc.subcore_barrier()

    base = wid * EPW

    def chunk_body(j, _):
        off = base + j * K
        pltpu.sync_copy(row_hbm.at[pl.ds(off, K)], row_v)
        pltpu.sync_copy(col_hbm.at[pl.ds(off, K)], col_v)
        pltpu.sync_copy(ew_hbm.at[pl.ds(off, K)], ew_v)
        pltpu.async_copy(y_hbm.at[row_v], xr_v, sem).wait()

        def scale_body(g, _):
            ew16 = ew_v[pl.ds(g * 16, 16)]
            for r in range(16):
                i = g * 16 + r
                xr_v[i] = xr_v[i] * ew16[r]
            return 0
        lax.fori_loop(0, K // 16, scale_body, 0)
        pltpu.sync_copy(xr_v, s_sp.at[col_v], add=True)
        return 0
    lax.fori_loop(0, CHUNKS, chunk_body, 0)

    plsc.subcore_barrier()
    pltpu.sync_copy(s_sp.at[pl.ds(sid * RPT, RPT)], zb)

    @pl.when(c == 0)
    def _():
        pltpu.sync_copy(zb, out0_hbm.at[pl.ds(sid * RPT, RPT)])

    @pl.when(c == 1)
    def _():
        pltpu.sync_copy(zb, out1_hbm.at[pl.ds(sid * RPT, RPT)])


B4 = 2048


def _out_body(s0_ref, s1_ref, dinv_ref, x_ref, att_ref, wcz_ref, bcz_ref,
              wlz_ref, blz_ref, wch_ref, bch_ref, wlh_ref, blh_ref, wout_ref,
              bout_ref, o_ref):
    dinv = dinv_ref[...]
    ssum = s0_ref[...] + s1_ref[...]
    s = dinv * ssum[:, :P] + (dinv * dinv) * x_ref[...]

    wlz = wlz_ref[...]
    wlh = wlh_ref[...]
    u_z = jnp.dot(wcz_ref[...], wlz[:HID], preferred_element_type=jnp.float32)
    c_z = jnp.dot(bcz_ref[...], wlz[:HID], preferred_element_type=jnp.float32) + blz_ref[...]
    u_h = jnp.dot(wch_ref[...], wlh[:HID], preferred_element_type=jnp.float32)
    c_h = jnp.dot(bch_ref[...], wlh[:HID], preferred_element_type=jnp.float32) + blh_ref[...]

    att = att_ref[...]
    ea = jnp.exp(att - jnp.max(att))
    probs = ea / jnp.sum(ea)

    acc = jnp.zeros((B4, HID), jnp.float32)
    for p in range(P):
        a = s[:, p:p + 1]
        z = jax.nn.sigmoid(a * u_z + c_z)
        ht = jnp.tanh(a * u_h + c_h)
        acc = acc + probs[0, p] * ((1.0 - z) * ht)
    h = jnp.where(acc > 0, acc, jnp.exp(jnp.minimum(acc, 0.0)) - 1.0)
    o_ref[...] = jnp.dot(h, wout_ref[...],
                         preferred_element_type=jnp.float32) + bout_ref[...]


_full = lambda i: (0, 0)

_readout = pl.pallas_call(
    _out_body,
    grid=(-(-N // B4),),
    in_specs=[
        pl.BlockSpec((B4, 16), lambda i: (i, 0)),
        pl.BlockSpec((B4, 16), lambda i: (i, 0)),
        pl.BlockSpec((B4, 1), lambda i: (i, 0)),
        pl.BlockSpec((B4, P), lambda i: (i, 0)),
        pl.BlockSpec((1, P), _full),
        pl.BlockSpec((1, HID), _full),
        pl.BlockSpec((1, HID), _full),
        pl.BlockSpec((2 * HID, HID), _full),
        pl.BlockSpec((1, HID), _full),
        pl.BlockSpec((1, HID), _full),
        pl.BlockSpec((1, HID), _full),
        pl.BlockSpec((2 * HID, HID), _full),
        pl.BlockSpec((1, HID), _full),
        pl.BlockSpec((HID, OUT), _full),
        pl.BlockSpec((1, OUT), _full),
    ],
    out_specs=pl.BlockSpec((B4, OUT), lambda i: (i, 0)),
    out_shape=jax.ShapeDtypeStruct((N, OUT), jnp.float32),
)


def kernel(x, edge_index, edge_weight, attention,
           W_conv_z, b_conv_z, W_lin_z, b_lin_z,
           W_conv_r, b_conv_r, W_lin_r, b_lin_r,
           W_conv_h, b_conv_h, W_lin_h, b_lin_h,
           W_out, b_out):
    row = edge_index[0]
    col = edge_index[1]
    pad = EP - E
    pad_idx = jnp.arange(pad, dtype=jnp.int32) % N
    row_p = jnp.concatenate([row, pad_idx])
    col_p = jnp.concatenate([col, pad_idx])
    ew_p = jnp.concatenate([edge_weight, jnp.zeros((pad,), jnp.float32)])

    deg0, deg1 = _deg_kernel(col_p, ew_p)
    dinv, y = _prescale(deg0, deg1, x)
    s0, s1 = _spmm_kernel(row_p, col_p, ew_p, y)
    return _readout(s0, s1, dinv, x, attention.reshape(1, P),
                    W_conv_z, b_conv_z.reshape(1, HID), W_lin_z,
                    b_lin_z.reshape(1, HID),
                    W_conv_h, b_conv_h.reshape(1, HID), W_lin_h,
                    b_lin_h.reshape(1, HID),
                    W_out, b_out.reshape(1, OUT))


# speedup vs baseline: 400.9620x; 2.3449x over previous
import functools

import jax
import jax.numpy as jnp
from jax import lax
from jax.experimental import pallas as pl
from jax.experimental.pallas import tpu as pltpu
from jax.experimental.pallas import tpu_sc as plsc

N = 50000
P = 12
HID = 32
OUT = 12
E = 800000

NC = 2
NS = 16
NW = NC * NS

K = 128
CHUNKS = -(-E // (NW * K))
EPW = CHUNKS * K
EP = NW * EPW
EG = EP // K

DBIG = 28
DITERS = CHUNKS // DBIG
SBIG = 14
SITERS = CHUNKS // SBIG

RPT = (-(-N // NS) + 15) // 16 * 16
N3 = NS * RPT

_mesh = plsc.VectorSubcoreMesh(core_axis_name="c", subcore_axis_name="s")


@functools.partial(
    pl.kernel,
    out_type=[jax.ShapeDtypeStruct((N3,), jnp.float32),
              jax.ShapeDtypeStruct((N3,), jnp.float32)],
    mesh=_mesh,
    scratch_types=[
        pltpu.VMEM((DBIG, K), jnp.int32),
        pltpu.VMEM((DBIG, K), jnp.float32),
        pltpu.VMEM((RPT,), jnp.float32),
        pltpu.VMEM_SHARED((N3,), jnp.float32),
        pltpu.SemaphoreType.DMA,
    ],
    compiler_params=pltpu.CompilerParams(use_tc_tiling_on_sc=False),
)
def _deg_kernel(col_hbm, ew_hbm, out0_hbm, out1_hbm, col_b, ew_b, zb, deg_sp,
                ssem):
    c = lax.axis_index("c")
    sid = lax.axis_index("s")
    wid = sid * NC + c

    def zero_body(i, _):
        zb[pl.ds(i * 16, 16)] = jnp.zeros((16,), jnp.float32)
        return 0
    lax.fori_loop(0, RPT // 16, zero_body, 0)
    pltpu.sync_copy(zb, deg_sp.at[pl.ds(sid * RPT, RPT)])
    plsc.subcore_barrier()

    gbase = wid * CHUNKS

    def iter_body(t, _):
        g0 = gbase + t * DBIG
        pltpu.sync_copy(col_hbm.at[pl.ds(g0, DBIG)], col_b)
        pltpu.sync_copy(ew_hbm.at[pl.ds(g0, DBIG)], ew_b)
        descs = [
            pltpu.async_copy(ew_b.at[k], deg_sp.at[col_b.at[k]], ssem,
                             add=True)
            for k in range(DBIG)
        ]
        for d in descs:
            d.wait()
        return 0
    lax.fori_loop(0, DITERS, iter_body, 0)

    plsc.subcore_barrier()
    pltpu.sync_copy(deg_sp.at[pl.ds(sid * RPT, RPT)], zb)

    @pl.when(c == 0)
    def _():
        pltpu.sync_copy(zb, out0_hbm.at[pl.ds(sid * RPT, RPT)])

    @pl.when(c == 1)
    def _():
        pltpu.sync_copy(zb, out1_hbm.at[pl.ds(sid * RPT, RPT)])


B2 = 2048


def _prescale_body(deg0_ref, deg1_ref, x_ref, dinv_ref, y_ref):
    deg = deg0_ref[...] + deg1_ref[...] + 1.0
    dinv = jnp.reshape(lax.rsqrt(deg), (B2, 1))
    dinv_ref[...] = dinv
    y_ref[...] = jnp.concatenate(
        [x_ref[...] * dinv, jnp.zeros((B2, 4), jnp.float32)], axis=1)


_prescale = pl.pallas_call(
    _prescale_body,
    grid=(-(-N // B2),),
    in_specs=[
        pl.BlockSpec((B2,), lambda i: (i,)),
        pl.BlockSpec((B2,), lambda i: (i,)),
        pl.BlockSpec((B2, P), lambda i: (i, 0)),
    ],
    out_specs=[
        pl.BlockSpec((B2, 1), lambda i: (i, 0)),
        pl.BlockSpec((B2, 16), lambda i: (i, 0)),
    ],
    out_shape=[
        jax.ShapeDtypeStruct((N, 1), jnp.float32),
        jax.ShapeDtypeStruct((N3, 16), jnp.float32),
    ],
)


@functools.partial(
    pl.kernel,
    out_type=[jax.ShapeDtypeStruct((N3, 16), jnp.float32),
              jax.ShapeDtypeStruct((N3, 16), jnp.float32)],
    mesh=_mesh,
    scratch_types=[
        pltpu.VMEM((SBIG, K), jnp.int32),
        pltpu.VMEM((SBIG, K), jnp.int32),
        pltpu.VMEM((SBIG, K), jnp.float32),
        pltpu.VMEM((SBIG * K, 16), jnp.float32),
        pltpu.VMEM((RPT // 4, 16), jnp.float32),
        pltpu.VMEM_SHARED((N3, 16), jnp.float32),
        pltpu.SemaphoreType.DMA,
        pltpu.SemaphoreType.DMA,
    ],
    compiler_params=pltpu.CompilerParams(use_tc_tiling_on_sc=False),
)
def _spmm_kernel(row_hbm, col_hbm, ew_hbm, y_hbm, out0_hbm, out1_hbm,
                 row_b, col_b, ew_b, xr, zb, s_sp, gsem, ssem):
    c = lax.axis_index("c")
    sid = lax.axis_index("s")
    wid = sid * NC + c

    def zero_body(i, _):
        zb[i] = jnp.zeros((16,), jnp.float32)
        return 0
    lax.fori_loop(0, RPT // 4, zero_body, 0)
    for q in range(4):
        pltpu.sync_copy(zb, s_sp.at[pl.ds(sid * RPT + q * (RPT // 4),
                                          RPT // 4)])
    plsc.subcore_barrier()

    gbase = wid * CHUNKS

    def iter_body(t, _):
        g0 = gbase + t * SBIG
        pltpu.sync_copy(row_hbm.at[pl.ds(g0, SBIG)], row_b)
        pltpu.sync_copy(col_hbm.at[pl.ds(g0, SBIG)], col_b)
        pltpu.sync_copy(ew_hbm.at[pl.ds(g0, SBIG)], ew_b)
        gds = [
            pltpu.async_copy(y_hbm.at[row_b.at[k]],
                             xr.at[pl.ds(k * K, K)], gsem)
            for k in range(SBIG)
        ]
        sds = []
        for k in range(SBIG):
            gds[k].wait()

            def scale_body(g, _, k=k):
                ew16 = ew_b[k, pl.ds(g * 16, 16)]
                for r in range(16):
                    i = k * K + g * 16 + r
                    xr[i] = xr[i] * ew16[r]
                return 0
            lax.fori_loop(0, K // 16, scale_body, 0)
            sds.append(
                pltpu.async_copy(xr.at[pl.ds(k * K, K)],
                                 s_sp.at[col_b.at[k]], ssem, add=True))
        for d in sds:
            d.wait()
        return 0
    lax.fori_loop(0, SITERS, iter_body, 0)

    plsc.subcore_barrier()
    for q in range(4):
        qoff = sid * RPT + q * (RPT // 4)
        pltpu.sync_copy(s_sp.at[pl.ds(qoff, RPT // 4)], zb)

        @pl.when(c == 0)
        def _():
            pltpu.sync_copy(zb, out0_hbm.at[pl.ds(qoff, RPT // 4)])

        @pl.when(c == 1)
        def _():
            pltpu.sync_copy(zb, out1_hbm.at[pl.ds(qoff, RPT // 4)])


B4 = 2048


def _out_body(s0_ref, s1_ref, dinv_ref, x_ref, att_ref, wcz_ref, bcz_ref,
              wlz_ref, blz_ref, wch_ref, bch_ref, wlh_ref, blh_ref, wout_ref,
              bout_ref, o_ref):
    dinv = dinv_ref[...]
    ssum = s0_ref[...] + s1_ref[...]
    s = dinv * ssum[:, :P] + (dinv * dinv) * x_ref[...]

    wlz = wlz_ref[...]
    wlh = wlh_ref[...]
    u_z = jnp.dot(wcz_ref[...], wlz[:HID], preferred_element_type=jnp.float32)
    c_z = jnp.dot(bcz_ref[...], wlz[:HID], preferred_element_type=jnp.float32) + blz_ref[...]
    u_h = jnp.dot(wch_ref[...], wlh[:HID], preferred_element_type=jnp.float32)
    c_h = jnp.dot(bch_ref[...], wlh[:HID], preferred_element_type=jnp.float32) + blh_ref[...]

    att = att_ref[...]
    ea = jnp.exp(att - jnp.max(att))
    probs = ea / jnp.sum(ea)

    acc = jnp.zeros((B4, HID), jnp.float32)
    for p in range(P):
        a = s[:, p:p + 1]
        z = jax.nn.sigmoid(a * u_z + c_z)
        ht = jnp.tanh(a * u_h + c_h)
        acc = acc + probs[0, p] * ((1.0 - z) * ht)
    h = jnp.where(acc > 0, acc, jnp.exp(jnp.minimum(acc, 0.0)) - 1.0)
    o_ref[...] = jnp.dot(h, wout_ref[...],
                         preferred_element_type=jnp.float32) + bout_ref[...]


_full = lambda i: (0, 0)

_readout = pl.pallas_call(
    _out_body,
    grid=(-(-N // B4),),
    in_specs=[
        pl.BlockSpec((B4, 16), lambda i: (i, 0)),
        pl.BlockSpec((B4, 16), lambda i: (i, 0)),
        pl.BlockSpec((B4, 1), lambda i: (i, 0)),
        pl.BlockSpec((B4, P), lambda i: (i, 0)),
        pl.BlockSpec((1, P), _full),
        pl.BlockSpec((1, HID), _full),
        pl.BlockSpec((1, HID), _full),
        pl.BlockSpec((2 * HID, HID), _full),
        pl.BlockSpec((1, HID), _full),
        pl.BlockSpec((1, HID), _full),
        pl.BlockSpec((1, HID), _full),
        pl.BlockSpec((2 * HID, HID), _full),
        pl.BlockSpec((1, HID), _full),
        pl.BlockSpec((HID, OUT), _full),
        pl.BlockSpec((1, OUT), _full),
    ],
    out_specs=pl.BlockSpec((B4, OUT), lambda i: (i, 0)),
    out_shape=jax.ShapeDtypeStruct((N, OUT), jnp.float32),
)


def kernel(x, edge_index, edge_weight, attention,
           W_conv_z, b_conv_z, W_lin_z, b_lin_z,
           W_conv_r, b_conv_r, W_lin_r, b_lin_r,
           W_conv_h, b_conv_h, W_lin_h, b_lin_h,
           W_out, b_out):
    row = edge_index[0]
    col = edge_index[1]
    pad = EP - E
    pad_idx = jnp.arange(pad, dtype=jnp.int32) % N
    row_p = jnp.concatenate([row, pad_idx]).reshape(EG, K)
    col_p = jnp.concatenate([col, pad_idx]).reshape(EG, K)
    ew_p = jnp.concatenate(
        [edge_weight, jnp.zeros((pad,), jnp.float32)]).reshape(EG, K)

    deg0, deg1 = _deg_kernel(col_p, ew_p)
    dinv, y = _prescale(deg0, deg1, x)
    s0, s1 = _spmm_kernel(row_p, col_p, ew_p, y)
    return _readout(s0, s1, dinv, x, attention.reshape(1, P),
                    W_conv_z, b_conv_z.reshape(1, HID), W_lin_z,
                    b_lin_z.reshape(1, HID),
                    W_conv_h, b_conv_h.reshape(1, HID), W_lin_h,
                    b_lin_h.reshape(1, HID),
                    W_out, b_out.reshape(1, OUT))


# speedup vs baseline: 525.0570x; 1.3095x over previous
import functools

import jax
import jax.numpy as jnp
from jax import lax
from jax.experimental import pallas as pl
from jax.experimental.pallas import tpu as pltpu
from jax.experimental.pallas import tpu_sc as plsc

N = 50000
P = 12
HID = 32
OUT = 12
E = 800000

NC = 2
NS = 16
NW = NC * NS

K = 128
CHUNKS = -(-E // (NW * K))
EPW = CHUNKS * K
EP = NW * EPW
EG = EP // K

DBIG = 28
DITERS = CHUNKS // DBIG
SBIG = 14
SITERS = CHUNKS // SBIG

RPT = (-(-N // NS) + 15) // 16 * 16
N3 = NS * RPT

_mesh = plsc.VectorSubcoreMesh(core_axis_name="c", subcore_axis_name="s")


@functools.partial(
    pl.kernel,
    out_type=[jax.ShapeDtypeStruct((N3,), jnp.float32),
              jax.ShapeDtypeStruct((N3,), jnp.float32)],
    mesh=_mesh,
    scratch_types=[
        pltpu.VMEM((DBIG, K), jnp.int32),
        pltpu.VMEM((DBIG, K), jnp.float32),
        pltpu.VMEM((RPT,), jnp.float32),
        pltpu.VMEM_SHARED((N3,), jnp.float32),
        pltpu.SemaphoreType.DMA,
    ],
    compiler_params=pltpu.CompilerParams(use_tc_tiling_on_sc=False),
)
def _deg_kernel(col_hbm, ew_hbm, out0_hbm, out1_hbm, col_b, ew_b, zb, deg_sp,
                ssem):
    c = lax.axis_index("c")
    sid = lax.axis_index("s")
    wid = sid * NC + c

    def zero_body(i, _):
        zb[pl.ds(i * 16, 16)] = jnp.zeros((16,), jnp.float32)
        return 0
    lax.fori_loop(0, RPT // 16, zero_body, 0)
    pltpu.sync_copy(zb, deg_sp.at[pl.ds(sid * RPT, RPT)])
    plsc.subcore_barrier()

    gbase = wid * CHUNKS

    def iter_body(t, _):
        g0 = gbase + t * DBIG
        pltpu.sync_copy(col_hbm.at[pl.ds(g0, DBIG)], col_b)
        pltpu.sync_copy(ew_hbm.at[pl.ds(g0, DBIG)], ew_b)
        descs = [
            pltpu.async_copy(ew_b.at[k], deg_sp.at[col_b.at[k]], ssem,
                             add=True)
            for k in range(DBIG)
        ]
        for d in descs:
            d.wait()
        return 0
    lax.fori_loop(0, DITERS, iter_body, 0)

    plsc.subcore_barrier()
    pltpu.sync_copy(deg_sp.at[pl.ds(sid * RPT, RPT)], zb)

    @pl.when(c == 0)
    def _():
        pltpu.sync_copy(zb, out0_hbm.at[pl.ds(sid * RPT, RPT)])

    @pl.when(c == 1)
    def _():
        pltpu.sync_copy(zb, out1_hbm.at[pl.ds(sid * RPT, RPT)])


B2 = 2048


def _prescale_body(deg0_ref, deg1_ref, x_ref, dinv_ref, y_ref):
    deg = deg0_ref[...] + deg1_ref[...] + 1.0
    dinv = jnp.reshape(lax.rsqrt(deg), (B2, 1))
    dinv_ref[...] = dinv
    y_ref[...] = jnp.concatenate(
        [x_ref[...] * dinv, jnp.zeros((B2, 4), jnp.float32)], axis=1)


_prescale = pl.pallas_call(
    _prescale_body,
    grid=(-(-N // B2),),
    in_specs=[
        pl.BlockSpec((B2,), lambda i: (i,)),
        pl.BlockSpec((B2,), lambda i: (i,)),
        pl.BlockSpec((B2, P), lambda i: (i, 0)),
    ],
    out_specs=[
        pl.BlockSpec((B2, 1), lambda i: (i, 0)),
        pl.BlockSpec((B2, 16), lambda i: (i, 0)),
    ],
    out_shape=[
        jax.ShapeDtypeStruct((N, 1), jnp.float32),
        jax.ShapeDtypeStruct((N3, 16), jnp.float32),
    ],
)


@functools.partial(
    pl.kernel,
    out_type=[jax.ShapeDtypeStruct((N3, 16), jnp.float32),
              jax.ShapeDtypeStruct((N3, 16), jnp.float32)],
    mesh=_mesh,
    scratch_types=[
        pltpu.VMEM((SBIG, K), jnp.int32),
        pltpu.VMEM((SBIG, K), jnp.int32),
        pltpu.VMEM((SBIG, K), jnp.float32),
        pltpu.VMEM((SBIG * K, 16), jnp.float32),
        pltpu.VMEM((RPT // 4, 16), jnp.float32),
        pltpu.VMEM_SHARED((N3, 16), jnp.float32),
        pltpu.SemaphoreType.DMA,
        pltpu.SemaphoreType.DMA,
    ],
    compiler_params=pltpu.CompilerParams(use_tc_tiling_on_sc=False),
)
def _spmm_kernel(row_hbm, col_hbm, ew_hbm, y_hbm, out0_hbm, out1_hbm,
                 row_b, col_b, ew_b, xr, zb, s_sp, gsem, ssem):
    c = lax.axis_index("c")
    sid = lax.axis_index("s")
    wid = sid * NC + c

    def zero_body(i, _):
        zb[i] = jnp.zeros((16,), jnp.float32)
        return 0
    lax.fori_loop(0, RPT // 4, zero_body, 0)
    for q in range(4):
        pltpu.sync_copy(zb, s_sp.at[pl.ds(sid * RPT + q * (RPT // 4),
                                          RPT // 4)])
    plsc.subcore_barrier()

    gbase = wid * CHUNKS

    def iter_body(t, _):
        g0 = gbase + t * SBIG
        pltpu.sync_copy(row_hbm.at[pl.ds(g0, SBIG)], row_b)
        pltpu.sync_copy(col_hbm.at[pl.ds(g0, SBIG)], col_b)
        pltpu.sync_copy(ew_hbm.at[pl.ds(g0, SBIG)], ew_b)
        gds = [
            pltpu.async_copy(y_hbm.at[row_b.at[k]],
                             xr.at[pl.ds(k * K, K)], gsem)
            for k in range(SBIG)
        ]
        sds = []
        for k in range(SBIG):
            gds[k].wait()

            def scale_body(g, _, k=k):
                ew16 = ew_b[k, pl.ds(g * 16, 16)]
                for r in range(16):
                    i = k * K + g * 16 + r
                    xr[i] = xr[i] * ew16[r]
                return 0
            lax.fori_loop(0, K // 16, scale_body, 0)
            sds.append(
                pltpu.async_copy(xr.at[pl.ds(k * K, K)],
                                 s_sp.at[col_b.at[k]], ssem, add=True))
        for d in sds:
            d.wait()
        return 0
    lax.fori_loop(0, SITERS, iter_body, 0)

    plsc.subcore_barrier()
    for q in range(4):
        qoff = sid * RPT + q * (RPT // 4)
        pltpu.sync_copy(s_sp.at[pl.ds(qoff, RPT // 4)], zb)

        @pl.when(c == 0)
        def _():
            pltpu.sync_copy(zb, out0_hbm.at[pl.ds(qoff, RPT // 4)])

        @pl.when(c == 1)
        def _():
            pltpu.sync_copy(zb, out1_hbm.at[pl.ds(qoff, RPT // 4)])


B4 = 2048


R4 = 2048 // 4


def _out_body(s0_ref, s1_ref, dinv_ref, x_ref, att_ref, wcz_ref, bcz_ref,
              wlz_ref, blz_ref, wch_ref, bch_ref, wlh_ref, blh_ref, wout_ref,
              bout_ref, o_ref):
    dinv = dinv_ref[...]
    ssum = s0_ref[...] + s1_ref[...]
    s = dinv * ssum[:, :P] + (dinv * dinv) * x_ref[...]

    wlz = wlz_ref[...]
    wlh = wlh_ref[...]
    u_z = jnp.dot(wcz_ref[...], wlz[:HID], preferred_element_type=jnp.float32)
    c_z = jnp.dot(bcz_ref[...], wlz[:HID], preferred_element_type=jnp.float32) + blz_ref[...]
    u_h = jnp.dot(wch_ref[...], wlh[:HID], preferred_element_type=jnp.float32)
    c_h = jnp.dot(bch_ref[...], wlh[:HID], preferred_element_type=jnp.float32) + blh_ref[...]

    att = att_ref[...]
    ea = jnp.exp(att - jnp.max(att))
    probs = ea / jnp.sum(ea)

    row_q = lax.broadcasted_iota(jnp.int32, (P, 4 * HID), 0)
    col_t = lax.broadcasted_iota(jnp.int32, (P, 4 * HID), 1) // HID
    uz_t = jnp.broadcast_to(jnp.tile(u_z, (1, 4)), (P, 4 * HID))
    uh_t = jnp.broadcast_to(jnp.tile(u_h, (1, 4)), (P, 4 * HID))
    cz4 = jnp.tile(c_z, (1, 4))
    ch4 = jnp.tile(c_h, (1, 4))
    tr_k = lax.broadcasted_iota(jnp.int32, (4 * HID, HID), 0) % HID
    tr_t = lax.broadcasted_iota(jnp.int32, (4 * HID, HID), 0) // HID
    tc_k = lax.broadcasted_iota(jnp.int32, (4 * HID, HID), 1)

    acc = jnp.zeros((B4, HID), jnp.float32)
    for p in range(0, P, 4):
        sel = row_q == p + col_t
        dmat = jnp.concatenate([jnp.where(sel, uz_t, 0.0),
                                jnp.where(sel, uh_t, 0.0)], axis=1)
        a = jnp.dot(s, dmat, preferred_element_type=jnp.float32)
        z = jax.nn.sigmoid(a[:, :4 * HID] + cz4)
        ht = jnp.tanh(a[:, 4 * HID:] + ch4)
        gated = (1.0 - z) * ht
        pr = jnp.concatenate(
            [jnp.full((HID, 1), probs[0, p + t]) for t in range(4)], axis=0)
        tmat = jnp.where(tr_k == tc_k, pr, 0.0)
        acc = acc + jnp.dot(gated, tmat, preferred_element_type=jnp.float32)
    h = jnp.where(acc > 0, acc, jnp.exp(jnp.minimum(acc, 0.0)) - 1.0)
    o_ref[...] = jnp.dot(h, wout_ref[...],
                         preferred_element_type=jnp.float32) + bout_ref[...]


_full = lambda i: (0, 0)

_readout = pl.pallas_call(
    _out_body,
    grid=(-(-N // B4),),
    in_specs=[
        pl.BlockSpec((B4, 16), lambda i: (i, 0)),
        pl.BlockSpec((B4, 16), lambda i: (i, 0)),
        pl.BlockSpec((B4, 1), lambda i: (i, 0)),
        pl.BlockSpec((B4, P), lambda i: (i, 0)),
        pl.BlockSpec((1, P), _full),
        pl.BlockSpec((1, HID), _full),
        pl.BlockSpec((1, HID), _full),
        pl.BlockSpec((2 * HID, HID), _full),
        pl.BlockSpec((1, HID), _full),
        pl.BlockSpec((1, HID), _full),
        pl.BlockSpec((1, HID), _full),
        pl.BlockSpec((2 * HID, HID), _full),
        pl.BlockSpec((1, HID), _full),
        pl.BlockSpec((HID, OUT), _full),
        pl.BlockSpec((1, OUT), _full),
    ],
    out_specs=pl.BlockSpec((B4, OUT), lambda i: (i, 0)),
    out_shape=jax.ShapeDtypeStruct((N, OUT), jnp.float32),
)


def kernel(x, edge_index, edge_weight, attention,
           W_conv_z, b_conv_z, W_lin_z, b_lin_z,
           W_conv_r, b_conv_r, W_lin_r, b_lin_r,
           W_conv_h, b_conv_h, W_lin_h, b_lin_h,
           W_out, b_out):
    row = edge_index[0]
    col = edge_index[1]
    pad = EP - E
    pad_idx = jnp.arange(pad, dtype=jnp.int32) % N
    row_p = jnp.concatenate([row, pad_idx]).reshape(EG, K)
    col_p = jnp.concatenate([col, pad_idx]).reshape(EG, K)
    ew_p = jnp.concatenate(
        [edge_weight, jnp.zeros((pad,), jnp.float32)]).reshape(EG, K)

    deg0, deg1 = _deg_kernel(col_p, ew_p)
    dinv, y = _prescale(deg0, deg1, x)
    s0, s1 = _spmm_kernel(row_p, col_p, ew_p, y)
    return _readout(s0, s1, dinv, x, attention.reshape(1, P),
                    W_conv_z, b_conv_z.reshape(1, HID), W_lin_z,
                    b_lin_z.reshape(1, HID),
                    W_conv_h, b_conv_h.reshape(1, HID), W_lin_h,
                    b_lin_h.reshape(1, HID),
                    W_out, b_out.reshape(1, OUT))


# speedup vs baseline: 561.2436x; 1.0689x over previous
import functools

import jax
import jax.numpy as jnp
from jax import lax
from jax.experimental import pallas as pl
from jax.experimental.pallas import tpu as pltpu
from jax.experimental.pallas import tpu_sc as plsc

N = 50000
P = 12
HID = 32
OUT = 12
E = 800000

NC = 2
NS = 16
NW = NC * NS

K = 128
CHUNKS = -(-E // (NW * K))
EPW = CHUNKS * K
EP = NW * EPW
EG = EP // K

DBIG = 49
DITERS = CHUNKS // DBIG
SBIG = 28
SITERS = CHUNKS // SBIG

RPT = (-(-N // NS) + 15) // 16 * 16
N3 = NS * RPT

_mesh = plsc.VectorSubcoreMesh(core_axis_name="c", subcore_axis_name="s")


@functools.partial(
    pl.kernel,
    out_type=[jax.ShapeDtypeStruct((N3,), jnp.float32),
              jax.ShapeDtypeStruct((N3,), jnp.float32)],
    mesh=_mesh,
    scratch_types=[
        pltpu.VMEM((DBIG, K), jnp.int32),
        pltpu.VMEM((DBIG, K), jnp.float32),
        pltpu.VMEM((RPT,), jnp.float32),
        pltpu.VMEM_SHARED((N3,), jnp.float32),
        pltpu.SemaphoreType.DMA,
    ],
    compiler_params=pltpu.CompilerParams(use_tc_tiling_on_sc=False),
)
def _deg_kernel(col_hbm, ew_hbm, out0_hbm, out1_hbm, col_b, ew_b, zb, deg_sp,
                ssem):
    c = lax.axis_index("c")
    sid = lax.axis_index("s")
    wid = sid * NC + c

    def zero_body(i, _):
        zb[pl.ds(i * 16, 16)] = jnp.zeros((16,), jnp.float32)
        return 0
    lax.fori_loop(0, RPT // 16, zero_body, 0)
    pltpu.sync_copy(zb, deg_sp.at[pl.ds(sid * RPT, RPT)])
    plsc.subcore_barrier()

    gbase = wid * CHUNKS

    def iter_body(t, _):
        g0 = gbase + t * DBIG
        pltpu.sync_copy(col_hbm.at[pl.ds(g0, DBIG)], col_b)
        pltpu.sync_copy(ew_hbm.at[pl.ds(g0, DBIG)], ew_b)
        descs = [
            pltpu.async_copy(ew_b.at[k], deg_sp.at[col_b.at[k]], ssem,
                             add=True)
            for k in range(DBIG)
        ]
        for d in descs:
            d.wait()
        return 0
    lax.fori_loop(0, DITERS, iter_body, 0)

    plsc.subcore_barrier()
    pltpu.sync_copy(deg_sp.at[pl.ds(sid * RPT, RPT)], zb)

    @pl.when(c == 0)
    def _():
        pltpu.sync_copy(zb, out0_hbm.at[pl.ds(sid * RPT, RPT)])

    @pl.when(c == 1)
    def _():
        pltpu.sync_copy(zb, out1_hbm.at[pl.ds(sid * RPT, RPT)])


B2 = 2048


def _prescale_body(deg0_ref, deg1_ref, x_ref, dinv_ref, y_ref):
    deg = deg0_ref[...] + deg1_ref[...] + 1.0
    dinv = jnp.reshape(lax.rsqrt(deg), (B2, 1))
    dinv_ref[...] = dinv
    y_ref[...] = jnp.concatenate(
        [x_ref[...] * dinv, jnp.zeros((B2, 4), jnp.float32)], axis=1)


_prescale = pl.pallas_call(
    _prescale_body,
    grid=(-(-N // B2),),
    in_specs=[
        pl.BlockSpec((B2,), lambda i: (i,)),
        pl.BlockSpec((B2,), lambda i: (i,)),
        pl.BlockSpec((B2, P), lambda i: (i, 0)),
    ],
    out_specs=[
        pl.BlockSpec((B2, 1), lambda i: (i, 0)),
        pl.BlockSpec((B2, 16), lambda i: (i, 0)),
    ],
    out_shape=[
        jax.ShapeDtypeStruct((N, 1), jnp.float32),
        jax.ShapeDtypeStruct((N3, 16), jnp.float32),
    ],
)


@functools.partial(
    pl.kernel,
    out_type=[jax.ShapeDtypeStruct((N3, 16), jnp.float32),
              jax.ShapeDtypeStruct((N3, 16), jnp.float32)],
    mesh=_mesh,
    scratch_types=[
        pltpu.VMEM((SBIG, K), jnp.int32),
        pltpu.VMEM((SBIG, K), jnp.int32),
        pltpu.VMEM((SBIG, K), jnp.float32),
        pltpu.VMEM((SBIG * K, 16), jnp.float32),
        pltpu.VMEM((RPT // 4, 16), jnp.float32),
        pltpu.VMEM_SHARED((N3, 16), jnp.float32),
        pltpu.SemaphoreType.DMA,
        pltpu.SemaphoreType.DMA,
    ],
    compiler_params=pltpu.CompilerParams(use_tc_tiling_on_sc=False),
)
def _spmm_kernel(row_hbm, col_hbm, ew_hbm, y_hbm, out0_hbm, out1_hbm,
                 row_b, col_b, ew_b, xr, zb, s_sp, gsem, ssem):
    c = lax.axis_index("c")
    sid = lax.axis_index("s")
    wid = sid * NC + c

    def zero_body(i, _):
        zb[i] = jnp.zeros((16,), jnp.float32)
        return 0
    lax.fori_loop(0, RPT // 4, zero_body, 0)
    for q in range(4):
        pltpu.sync_copy(zb, s_sp.at[pl.ds(sid * RPT + q * (RPT // 4),
                                          RPT // 4)])
    plsc.subcore_barrier()

    gbase = wid * CHUNKS

    def iter_body(t, _):
        g0 = gbase + t * SBIG
        pltpu.sync_copy(row_hbm.at[pl.ds(g0, SBIG)], row_b)
        pltpu.sync_copy(col_hbm.at[pl.ds(g0, SBIG)], col_b)
        pltpu.sync_copy(ew_hbm.at[pl.ds(g0, SBIG)], ew_b)
        gds = [
            pltpu.async_copy(y_hbm.at[row_b.at[k]],
                             xr.at[pl.ds(k * K, K)], gsem)
            for k in range(SBIG)
        ]
        sds = []
        for k in range(SBIG):
            gds[k].wait()

            def scale_body(g, _, k=k):
                ew16 = ew_b[k, pl.ds(g * 16, 16)]
                for r in range(16):
                    i = k * K + g * 16 + r
                    xr[i] = xr[i] * ew16[r]
                return 0
            lax.fori_loop(0, K // 16, scale_body, 0)
            sds.append(
                pltpu.async_copy(xr.at[pl.ds(k * K, K)],
                                 s_sp.at[col_b.at[k]], ssem, add=True))
        for d in sds:
            d.wait()
        return 0
    lax.fori_loop(0, SITERS, iter_body, 0)

    plsc.subcore_barrier()
    for q in range(4):
        qoff = sid * RPT + q * (RPT // 4)
        pltpu.sync_copy(s_sp.at[pl.ds(qoff, RPT // 4)], zb)

        @pl.when(c == 0)
        def _():
            pltpu.sync_copy(zb, out0_hbm.at[pl.ds(qoff, RPT // 4)])

        @pl.when(c == 1)
        def _():
            pltpu.sync_copy(zb, out1_hbm.at[pl.ds(qoff, RPT // 4)])


B4 = 2048


R4 = 2048 // 4


def _out_body(s0_ref, s1_ref, dinv_ref, x_ref, att_ref, wcz_ref, bcz_ref,
              wlz_ref, blz_ref, wch_ref, bch_ref, wlh_ref, blh_ref, wout_ref,
              bout_ref, o_ref):
    dinv = dinv_ref[...]
    ssum = s0_ref[...] + s1_ref[...]
    s = dinv * ssum[:, :P] + (dinv * dinv) * x_ref[...]

    wlz = wlz_ref[...]
    wlh = wlh_ref[...]
    u_z = jnp.dot(wcz_ref[...], wlz[:HID], preferred_element_type=jnp.float32)
    c_z = jnp.dot(bcz_ref[...], wlz[:HID], preferred_element_type=jnp.float32) + blz_ref[...]
    u_h = jnp.dot(wch_ref[...], wlh[:HID], preferred_element_type=jnp.float32)
    c_h = jnp.dot(bch_ref[...], wlh[:HID], preferred_element_type=jnp.float32) + blh_ref[...]

    att = att_ref[...]
    ea = jnp.exp(att - jnp.max(att))
    probs = ea / jnp.sum(ea)

    row_q = lax.broadcasted_iota(jnp.int32, (P, 4 * HID), 0)
    col_t = lax.broadcasted_iota(jnp.int32, (P, 4 * HID), 1) // HID
    uz_t = jnp.broadcast_to(jnp.tile(u_z, (1, 4)), (P, 4 * HID))
    uh_t = jnp.broadcast_to(jnp.tile(u_h, (1, 4)), (P, 4 * HID))
    cz4 = jnp.tile(c_z, (1, 4))
    ch4 = jnp.tile(c_h, (1, 4))
    tr_k = lax.broadcasted_iota(jnp.int32, (4 * HID, HID), 0) % HID
    tr_t = lax.broadcasted_iota(jnp.int32, (4 * HID, HID), 0) // HID
    tc_k = lax.broadcasted_iota(jnp.int32, (4 * HID, HID), 1)

    acc = jnp.zeros((B4, HID), jnp.float32)
    for p in range(0, P, 4):
        sel = row_q == p + col_t
        dmat = jnp.concatenate([jnp.where(sel, uz_t, 0.0),
                                jnp.where(sel, uh_t, 0.0)], axis=1)
        a = jnp.dot(s, dmat, preferred_element_type=jnp.float32)
        z = jax.nn.sigmoid(a[:, :4 * HID] + cz4)
        ht = jnp.tanh(a[:, 4 * HID:] + ch4)
        gated = (1.0 - z) * ht
        pr = jnp.concatenate(
            [jnp.full((HID, 1), probs[0, p + t]) for t in range(4)], axis=0)
        tmat = jnp.where(tr_k == tc_k, pr, 0.0)
        acc = acc + jnp.dot(gated, tmat, preferred_element_type=jnp.float32)
    h = jnp.where(acc > 0, acc, jnp.exp(jnp.minimum(acc, 0.0)) - 1.0)
    o_ref[...] = jnp.dot(h, wout_ref[...],
                         preferred_element_type=jnp.float32) + bout_ref[...]


_full = lambda i: (0, 0)

_readout = pl.pallas_call(
    _out_body,
    grid=(-(-N // B4),),
    in_specs=[
        pl.BlockSpec((B4, 16), lambda i: (i, 0)),
        pl.BlockSpec((B4, 16), lambda i: (i, 0)),
        pl.BlockSpec((B4, 1), lambda i: (i, 0)),
        pl.BlockSpec((B4, P), lambda i: (i, 0)),
        pl.BlockSpec((1, P), _full),
        pl.BlockSpec((1, HID), _full),
        pl.BlockSpec((1, HID), _full),
        pl.BlockSpec((2 * HID, HID), _full),
        pl.BlockSpec((1, HID), _full),
        pl.BlockSpec((1, HID), _full),
        pl.BlockSpec((1, HID), _full),
        pl.BlockSpec((2 * HID, HID), _full),
        pl.BlockSpec((1, HID), _full),
        pl.BlockSpec((HID, OUT), _full),
        pl.BlockSpec((1, OUT), _full),
    ],
    out_specs=pl.BlockSpec((B4, OUT), lambda i: (i, 0)),
    out_shape=jax.ShapeDtypeStruct((N, OUT), jnp.float32),
)


def kernel(x, edge_index, edge_weight, attention,
           W_conv_z, b_conv_z, W_lin_z, b_lin_z,
           W_conv_r, b_conv_r, W_lin_r, b_lin_r,
           W_conv_h, b_conv_h, W_lin_h, b_lin_h,
           W_out, b_out):
    row = edge_index[0]
    col = edge_index[1]
    pad = EP - E
    pad_idx = jnp.arange(pad, dtype=jnp.int32) % N
    row_p = jnp.concatenate([row, pad_idx]).reshape(EG, K)
    col_p = jnp.concatenate([col, pad_idx]).reshape(EG, K)
    ew_p = jnp.concatenate(
        [edge_weight, jnp.zeros((pad,), jnp.float32)]).reshape(EG, K)

    deg0, deg1 = _deg_kernel(col_p, ew_p)
    dinv, y = _prescale(deg0, deg1, x)
    s0, s1 = _spmm_kernel(row_p, col_p, ew_p, y)
    return _readout(s0, s1, dinv, x, attention.reshape(1, P),
                    W_conv_z, b_conv_z.reshape(1, HID), W_lin_z,
                    b_lin_z.reshape(1, HID),
                    W_conv_h, b_conv_h.reshape(1, HID), W_lin_h,
                    b_lin_h.reshape(1, HID),
                    W_out, b_out.reshape(1, OUT))
